# baseline (speedup 1.0000x reference)
import functools

import jax
import jax.numpy as jnp
from jax import lax
from jax.experimental import pallas as pl
from jax.experimental.pallas import tpu as pltpu
from jax.experimental.pallas import tpu_sc as plsc

KTOP = 64
ROWS = 128
N = 32768
L = 16
NV = N // L
NBINS = 256
WORKERS = 32
ROWS_PER_W = ROWS // WORKERS
U = 8

_mesh = plsc.VectorSubcoreMesh(core_axis_name="c", subcore_axis_name="s")


@functools.partial(
    pl.kernel,
    mesh=_mesh,
    compiler_params=pltpu.CompilerParams(needs_layout_passes=False),
    out_type=jax.ShapeDtypeStruct((ROWS, N), jnp.float32),
    scratch_types=[
        pltpu.VMEM((N,), jnp.float32),
        pltpu.VMEM((N,), jnp.float32),
        pltpu.VMEM((N,), jnp.int32),
        pltpu.VMEM((2 * L,), jnp.int32),
        pltpu.VMEM((NBINS * L,), jnp.int32),
        pltpu.SemaphoreType.DMA,
        pltpu.SemaphoreType.DMA,
        pltpu.SemaphoreType.DMA,
        pltpu.SemaphoreType.DMA,
    ],
)
def _topk_sc(x_hbm, out_hbm, xbufa, xbufb, cand, cbuf, hist,
             lsema, lsemb, ssema, ssemb):
    wid = lax.axis_index("s") * 2 + lax.axis_index("c")
    lane = lax.iota(jnp.int32, L)
    ones = jnp.ones((L,), jnp.int32)
    zeros_i = jnp.zeros((L,), jnp.int32)

    def zkey(bits):
        m = bits >> 31
        return bits ^ (m & 0x7FFFFFFF)

    def clear_hist():
        def body(i, carry):
            for j in range(U):
                hist[pl.ds((i * U + j) * L, L)] = zeros_i
            return carry
        lax.fori_loop(0, NBINS // U, body, 0)

    def scan_level(a_in, b_start):
        def bin_sum(b):
            return jnp.sum(hist[pl.ds(b * L, L)])

        def cond(st):
            b, csum, s = st
            return a_in + csum + s < KTOP

        def body(st):
            b, csum, s = st
            return b - 1, csum + s, bin_sum(b - 1)

        b, csum, s = lax.while_loop(
            cond, body, (b_start, jnp.int32(0), bin_sum(b_start)))
        return b, a_in + csum, s

    def do_row(xbuf, r, prefetch):
        def ld(i):
            return plsc.bitcast(xbuf[pl.ds(i * L, L)], jnp.int32)

        clear_hist()

        def hist1_body(i, mx):
            xs = [ld(i * U + j) for j in range(U)]
            zs = [zkey(xv) for xv in xs]
            idxs = [((((z >> 24) + 128) << 4) | lane) for z in zs]
            for z in zs:
                mx = jnp.maximum(mx, z)
            for idx in idxs:
                plsc.addupdate_scatter(hist, [idx], ones)
            return mx
        mx = lax.fori_loop(0, NV // U, hist1_body,
                           jnp.full((L,), -(2 ** 31), jnp.int32))
        b1_start = (jnp.max(mx) >> 24) + 128

        prefetch()

        b1, above, cnt = scan_level(jnp.int32(0), b1_start)

        b1r = b1 - 128
        lane_base = lane << 11

        def extract1_body(i, st):
            cnt_vec, dmx = st
            xs = [ld(i * U + j) for j in range(U)]
            zs = [zkey(xv) for xv in xs]
            ms = [(z >> 24) == b1r for z in zs]
            for z, m in zip(zs, ms):
                plsc.store_scatter(cand, [lane_base + cnt_vec], z, mask=m)
                cnt_vec = cnt_vec + m.astype(jnp.int32)
                dmx = jnp.maximum(dmx, jnp.where(m, (z >> 16) & 0xFF, zeros_i))
            return cnt_vec, dmx
        cnt_vec, dmx = lax.fori_loop(0, NV // U, extract1_body,
                                     (zeros_i, zeros_i))
        max_cnt = jnp.max(cnt_vec)
        d2_start = jnp.max(dmx)

        clear_hist()

        def ragged(i):
            zc = plsc.load_gather(cand, [lane_base + i])
            valid = cnt_vec > (zeros_i + i)
            return zc, valid

        def hist2_body(i, carry):
            zc, valid = ragged(i)
            d = (zc >> 16) & 0xFF
            plsc.addupdate_scatter(hist, [(d << 4) | lane], ones, mask=valid)
            return carry
        lax.fori_loop(0, max_cnt, hist2_body, 0)
        b2, above, cnt2 = scan_level(above, d2_start)

        need = KTOP - above

        def sort_path(_):
            def collect(i, off):
                zc, valid = ragged(i)
                m = valid & (((zc >> 16) & 0xFF) == b2)
                mi = m.astype(jnp.int32)
                prefix = plsc.cumsum(mi) - mi
                plsc.store_scatter(cbuf, [off + prefix], zc, mask=m)
                return off + plsc.all_reduce_population_count(m)
            lax.fori_loop(0, max_cnt, collect, zeros_i)
            zc = cbuf[pl.ds(0, L)]
            sk, _, _ = plsc.sort_key_val(zc, zc, mask=lane < cnt2,
                                         descending=True)
            t = jnp.max(jnp.where(lane == (need - 1), sk, -(2 ** 31)))
            tvv = zeros_i + t
            n_gt = jnp.max(plsc.all_reduce_population_count(
                (sk > tvv) & (lane < cnt2)))
            n_eq = jnp.max(plsc.all_reduce_population_count(
                (sk == tvv) & (lane < cnt2)))
            return t, above + n_gt, n_eq

        def bisect_path(_):
            p0 = (b1r << 24) | (b2 << 16)

            def count_eq(qv, shift):
                def body(i, acc):
                    zc, valid = ragged(i)
                    m = (valid & (((zc >> 16) & 0xFF) == b2)
                         & ((zc >> shift) == qv))
                    return acc + plsc.all_reduce_population_count(m)
                return jnp.max(lax.fori_loop(0, max_cnt, body, zeros_i))

            t, abv = p0, above
            for p in range(15, -1, -1):
                q = (t >> p) | 1
                s = count_eq(zeros_i + q, p)
                take = abv + s >= KTOP
                t = lax.select(take, t | (1 << p), t)
                abv = lax.select(take, abv, abv + s)
            n_eq = count_eq(zeros_i + t, 0)
            return t, abv, n_eq

        t, above, cnt = lax.cond(cnt2 <= L, sort_path, bisect_path, 0)
        tie_keep = KTOP - above
        tv = zeros_i + t

        tpos = zeros_i + jnp.maximum(t, 1)

        def fast_path(carry):
            def body(i, carry):
                xs = [ld(i * U + j) for j in range(U)]
                zs = [zkey(xv) for xv in xs]
                outs = [jnp.where(z >= tpos, xv, zeros_i)
                        for xv, z in zip(xs, zs)]
                for j in range(U):
                    xbuf[pl.ds((i * U + j) * L, L)] = plsc.bitcast(
                        outs[j], jnp.float32)
                return carry
            return lax.fori_loop(0, NV // U, body, carry)

        def slow_path(carry):
            ev = zeros_i + tie_keep

            def body(i, seen):
                xv = ld(i)
                z = zkey(xv)
                eq = z == tv
                ei = eq.astype(jnp.int32)
                prefix = plsc.cumsum(ei) - ei
                keep = ((z > tv) | (eq & ((seen + prefix) < ev))) & (xv > 0)
                xbuf[pl.ds(i * L, L)] = plsc.bitcast(
                    jnp.where(keep, xv, zeros_i), jnp.float32)
                return seen + plsc.all_reduce_population_count(eq)
            lax.fori_loop(0, NV, body, zeros_i)
            return 0

        lax.cond(tie_keep == cnt, fast_path, slow_path, 0)

    r0 = wid * ROWS_PER_W
    bufs = [xbufa, xbufb]
    lsems = [lsema, lsemb]
    ssems = [ssema, ssemb]
    load_h = [None] * ROWS_PER_W
    store_h = [None] * ROWS_PER_W
    load_h[0] = pltpu.async_copy(x_hbm.at[r0], bufs[0], lsems[0])
    for rr in range(ROWS_PER_W):
        buf = bufs[rr % 2]
        load_h[rr].wait()

        def prefetch(rr=rr):
            if rr + 1 < ROWS_PER_W:
                if rr >= 1:
                    store_h[rr - 1].wait()
                load_h[rr + 1] = pltpu.async_copy(
                    x_hbm.at[r0 + rr + 1], bufs[(rr + 1) % 2],
                    lsems[(rr + 1) % 2])

        do_row(buf, r0 + rr, prefetch)
        store_h[rr] = pltpu.async_copy(buf, out_hbm.at[r0 + rr],
                                       ssems[rr % 2])
    store_h[ROWS_PER_W - 2].wait()
    store_h[ROWS_PER_W - 1].wait()


def kernel(x):
    return _topk_sc(x)


# speedup vs baseline: 1.0150x; 1.0150x over previous
import functools

import jax
import jax.numpy as jnp
from jax import lax
from jax.experimental import pallas as pl
from jax.experimental.pallas import tpu as pltpu
from jax.experimental.pallas import tpu_sc as plsc

KTOP = 64
ROWS = 128
N = 32768
L = 16
NV = N // L
NBINS = 256
WORKERS = 32
ROWS_PER_W = ROWS // WORKERS
U = 8

_mesh = plsc.VectorSubcoreMesh(core_axis_name="c", subcore_axis_name="s")


@functools.partial(
    pl.kernel,
    mesh=_mesh,
    compiler_params=pltpu.CompilerParams(needs_layout_passes=False),
    out_type=jax.ShapeDtypeStruct((ROWS, N), jnp.float32),
    scratch_types=[
        pltpu.VMEM((N,), jnp.float32),
        pltpu.VMEM((N,), jnp.float32),
        pltpu.VMEM((N + 2 * L,), jnp.int32),
        pltpu.VMEM((NBINS * L,), jnp.int32),
        pltpu.SemaphoreType.DMA,
        pltpu.SemaphoreType.DMA,
        pltpu.SemaphoreType.DMA,
        pltpu.SemaphoreType.DMA,
    ],
)
def _topk_sc(x_hbm, out_hbm, xbufa, xbufb, cand, hist,
             lsema, lsemb, ssema, ssemb):
    wid = lax.axis_index("s") * 2 + lax.axis_index("c")
    lane = lax.iota(jnp.int32, L)
    ones = jnp.ones((L,), jnp.int32)
    zeros_i = jnp.zeros((L,), jnp.int32)

    def zkey(bits):
        m = bits >> 31
        return bits ^ (m & 0x7FFFFFFF)

    def clear_hist():
        def body(i, carry):
            for j in range(U):
                hist[pl.ds((i * U + j) * L, L)] = zeros_i
            return carry
        lax.fori_loop(0, NBINS // U, body, 0)

    def scan_level(a_in, b_start):
        def bin_sum(b):
            return jnp.sum(hist[pl.ds(b * L, L)])

        def cond(st):
            b, csum, s = st
            return a_in + csum + s < KTOP

        def body(st):
            b, csum, s = st
            return b - 1, csum + s, bin_sum(b - 1)

        b, csum, s = lax.while_loop(
            cond, body, (b_start, jnp.int32(0), bin_sum(b_start)))
        return b, a_in + csum, s

    def do_row(xbuf, r, prefetch):
        def ld(i):
            return plsc.bitcast(xbuf[pl.ds(i * L, L)], jnp.int32)

        clear_hist()

        def hist1_body(i, mx):
            xs = [ld(i * U + j) for j in range(U)]
            zs = [zkey(xv) for xv in xs]
            idxs = [((((z >> 24) + 128) << 4) | lane) for z in zs]
            for z in zs:
                mx = jnp.maximum(mx, z)
            for idx in idxs:
                plsc.addupdate_scatter(hist, [idx], ones)
            return mx
        mx = lax.fori_loop(0, NV // U, hist1_body,
                           jnp.full((L,), -(2 ** 31), jnp.int32))
        b1_start = (jnp.max(mx) >> 24) + 128

        prefetch()

        b1, above, cnt = scan_level(jnp.int32(0), b1_start)

        b1r = b1 - 128

        def extract1_body(i, st):
            off, dmx = st
            xs = [ld(i * U + j) for j in range(U)]
            zs = [zkey(xv) for xv in xs]
            ms = [(z >> 24) == b1r for z in zs]
            mis = [m.astype(jnp.int32) for m in ms]
            prefixes = [plsc.cumsum(mi) - mi for mi in mis]
            pcs = [plsc.all_reduce_population_count(m) for m in ms]
            for z, m in zip(zs, ms):
                dmx = jnp.maximum(dmx, jnp.where(m, (z >> 16) & 0xFF, zeros_i))
            for z, m, prefix, pc in zip(zs, ms, prefixes, pcs):
                plsc.store_scatter(cand, [off + prefix], z, mask=m)
                off = off + pc
            return off, dmx
        off, dmx = lax.fori_loop(0, NV // U, extract1_body, (zeros_i, zeros_i))
        c = jnp.max(off)
        d2_start = jnp.max(dmx)

        clear_hist()
        nv = (c + L - 1) >> 4

        def hist2_body(i, carry):
            zc = cand[pl.ds(i * L, L)]
            tail = (i * L + lane) < c
            d = (zc >> 16) & 0xFF
            plsc.addupdate_scatter(hist, [(d << 4) | lane], ones, mask=tail)
            return carry
        lax.fori_loop(0, nv, hist2_body, 0)
        b2, above, cnt = scan_level(above, d2_start)

        def extract2_body(i, off):
            zc = cand[pl.ds(i * L, L)]
            tail = (i * L + lane) < c
            m = tail & (((zc >> 16) & 0xFF) == b2)
            mi = m.astype(jnp.int32)
            prefix = plsc.cumsum(mi) - mi
            plsc.store_scatter(cand, [off + prefix], zc, mask=m)
            return off + plsc.all_reduce_population_count(m)
        c2 = jnp.max(lax.fori_loop(0, nv, extract2_body, zeros_i))

        need = KTOP - above

        def sort_path(_):
            zc = cand[pl.ds(0, L)]
            sk, _, _ = plsc.sort_key_val(zc, zc, mask=lane < c2,
                                         descending=True)
            t = jnp.max(jnp.where(lane == (need - 1), sk, -(2 ** 31)))
            tvv = zeros_i + t
            n_gt = jnp.max(plsc.all_reduce_population_count(
                (sk > tvv) & (lane < c2)))
            n_eq = jnp.max(plsc.all_reduce_population_count(
                (sk == tvv) & (lane < c2)))
            return t, above + n_gt, n_eq

        def bisect_path(_):
            nv2 = (c2 + L - 1) >> 4
            p0 = (b1r << 24) | (b2 << 16)

            def count_eq(qv, shift):
                def body(i, acc):
                    zc = cand[pl.ds(i * L, L)]
                    tail = (i * L + lane) < c2
                    m = tail & ((zc >> shift) == qv)
                    return acc + plsc.all_reduce_population_count(m)
                return jnp.max(lax.fori_loop(0, nv2, body, zeros_i))

            t, abv = p0, above
            for p in range(15, -1, -1):
                q = (t >> p) | 1
                s = count_eq(zeros_i + q, p)
                take = abv + s >= KTOP
                t = lax.select(take, t | (1 << p), t)
                abv = lax.select(take, abv, abv + s)
            n_eq = count_eq(zeros_i + t, 0)
            return t, abv, n_eq

        t, above, cnt = lax.cond(c2 <= L, sort_path, bisect_path, 0)
        tie_keep = KTOP - above
        tv = zeros_i + t

        tpos = zeros_i + jnp.maximum(t, 1)

        def fast_path(carry):
            def body(i, carry):
                xs = [ld(i * U + j) for j in range(U)]
                zs = [zkey(xv) for xv in xs]
                outs = [jnp.where(z >= tpos, xv, zeros_i)
                        for xv, z in zip(xs, zs)]
                for j in range(U):
                    xbuf[pl.ds((i * U + j) * L, L)] = plsc.bitcast(
                        outs[j], jnp.float32)
                return carry
            return lax.fori_loop(0, NV // U, body, carry)

        def slow_path(carry):
            ev = zeros_i + tie_keep

            def body(i, seen):
                xv = ld(i)
                z = zkey(xv)
                eq = z == tv
                ei = eq.astype(jnp.int32)
                prefix = plsc.cumsum(ei) - ei
                keep = ((z > tv) | (eq & ((seen + prefix) < ev))) & (xv > 0)
                xbuf[pl.ds(i * L, L)] = plsc.bitcast(
                    jnp.where(keep, xv, zeros_i), jnp.float32)
                return seen + plsc.all_reduce_population_count(eq)
            lax.fori_loop(0, NV, body, zeros_i)
            return 0

        lax.cond(tie_keep == cnt, fast_path, slow_path, 0)

    r0 = wid * ROWS_PER_W
    bufs = [xbufa, xbufb]
    lsems = [lsema, lsemb]
    ssems = [ssema, ssemb]
    load_h = [None] * ROWS_PER_W
    store_h = [None] * ROWS_PER_W
    load_h[0] = pltpu.async_copy(x_hbm.at[r0], bufs[0], lsems[0])
    for rr in range(ROWS_PER_W):
        buf = bufs[rr % 2]
        load_h[rr].wait()

        def prefetch(rr=rr):
            if rr + 1 < ROWS_PER_W:
                if rr >= 1:
                    store_h[rr - 1].wait()
                load_h[rr + 1] = pltpu.async_copy(
                    x_hbm.at[r0 + rr + 1], bufs[(rr + 1) % 2],
                    lsems[(rr + 1) % 2])

        do_row(buf, r0 + rr, prefetch)
        store_h[rr] = pltpu.async_copy(buf, out_hbm.at[r0 + rr],
                                       ssems[rr % 2])
    store_h[ROWS_PER_W - 2].wait()
    store_h[ROWS_PER_W - 1].wait()


def kernel(x):
    return _topk_sc(x)
